# Initial kernel scaffold; baseline (speedup 1.0000x reference)
import functools

import jax
import jax.numpy as jnp
from jax import lax
from jax.experimental import pallas as pl
from jax.experimental.pallas import tpu as pltpu
from jax.experimental.pallas import tpu_sc as plsc

N_NODES = 10000
N_EDGES = 320000
D = 128
DP = 144
NC = 2
NS = 16
E_PER = N_EDGES // (NC * NS)
R_PER = N_NODES // NS
ROW_BLK = 1000


def _proj_body(x_ref, ws_ref, wd_ref, aw_ref, z_ref, s2_ref):
    xb = x_ref[...]
    zs = lax.dot_general(xb, ws_ref[...], (((1,), (1,)), ((), ())),
                         preferred_element_type=jnp.float32)
    zd = lax.dot_general(xb, wd_ref[...], (((1,), (1,)), ((), ())),
                         preferred_element_type=jnp.float32)
    z_ref[...] = zs
    a = aw_ref[...]
    s_src = lax.dot_general(a[:, :D], zs, (((1,), (1,)), ((), ())),
                            preferred_element_type=jnp.float32)
    s_dst = lax.dot_general(a[:, D:], zd, (((1,), (1,)), ((), ())),
                            preferred_element_type=jnp.float32)
    s2_ref[...] = jnp.concatenate([s_src, s_dst], axis=0)


def _project(x, W_src, W_dst, attn_w):
    return pl.pallas_call(
        _proj_body,
        grid=(N_NODES // ROW_BLK,),
        in_specs=[
            pl.BlockSpec((ROW_BLK, D), lambda i: (i, 0)),
            pl.BlockSpec((D, D), lambda i: (0, 0)),
            pl.BlockSpec((D, D), lambda i: (0, 0)),
            pl.BlockSpec((1, 2 * D), lambda i: (0, 0)),
        ],
        out_specs=[
            pl.BlockSpec((ROW_BLK, D), lambda i: (i, 0)),
            pl.BlockSpec((2, ROW_BLK), lambda i: (0, i)),
        ],
        out_shape=[
            jax.ShapeDtypeStruct((N_NODES, D), jnp.float32),
            jax.ShapeDtypeStruct((2, N_NODES), jnp.float32),
        ],
    )(x, W_src, W_dst, attn_w)


def _edge_body(z_hbm, s2_hbm, ei_hbm, zeros_hbm, hp_hbm,
               src_v, dst_v, ssrc_v, sdst_v, rows_v, stage_v, h_sh, sem):
    c = lax.axis_index("c")
    s = lax.axis_index("s")
    base = (c * NS + s) * E_PER
    pltpu.sync_copy(ei_hbm.at[0, pl.ds(base, E_PER)], src_v)
    pltpu.sync_copy(ei_hbm.at[1, pl.ds(base, E_PER)], dst_v)
    pltpu.sync_copy(s2_hbm.at[0], ssrc_v)
    pltpu.sync_copy(s2_hbm.at[1], sdst_v)
    rbase = s * R_PER
    pltpu.sync_copy(zeros_hbm.at[pl.ds(rbase, R_PER)],
                    h_sh.at[pl.ds(rbase, R_PER)])
    zero16 = jnp.zeros((16,), jnp.float32)
    for j in range(16):
        stage_v[j, pl.ds(D, 16)] = zero16
    plsc.subcore_barrier()

    iota16 = lax.iota(jnp.int32, 16)
    col128 = jnp.full((16,), D, jnp.int32)

    def body(g, carry):
        eb = g * 16
        srcv = src_v[pl.ds(eb, 16)]
        dstv = dst_v[pl.ds(eb, 16)]
        sv = plsc.load_gather(ssrc_v, [srcv])
        dv = plsc.load_gather(sdst_v, [dstv])
        ev = sv + dv
        ev = jnp.where(ev >= 0.0, ev, ev * 0.01)
        exv = jnp.exp(ev)
        pltpu.async_copy(z_hbm.at[srcv], rows_v, sem).wait()
        for col in range(D):
            cidx = jnp.full((16,), col, jnp.int32)
            v = plsc.load_gather(rows_v, [iota16, cidx])
            plsc.store_scatter(stage_v, [iota16, cidx], v * exv)
        plsc.store_scatter(stage_v, [iota16, col128], exv)
        pltpu.sync_copy(stage_v, h_sh.at[dstv], add=True)
        return carry

    lax.fori_loop(0, E_PER // 16, body, 0)
    plsc.subcore_barrier()
    pltpu.sync_copy(h_sh.at[pl.ds(rbase, R_PER)],
                    hp_hbm.at[c, pl.ds(rbase, R_PER)])


@functools.partial(
    pl.kernel,
    out_type=jax.ShapeDtypeStruct((NC, N_NODES, DP), jnp.float32),
    mesh=plsc.VectorSubcoreMesh(core_axis_name="c", subcore_axis_name="s"),
    scratch_types=[
        pltpu.VMEM((E_PER,), jnp.int32),
        pltpu.VMEM((E_PER,), jnp.int32),
        pltpu.VMEM((N_NODES,), jnp.float32),
        pltpu.VMEM((N_NODES,), jnp.float32),
        pltpu.VMEM((16, D), jnp.float32),
        pltpu.VMEM((16, DP), jnp.float32),
        pltpu.VMEM_SHARED((N_NODES, DP), jnp.float32),
        pltpu.SemaphoreType.DMA,
    ],
)
def _edge_pass(z_hbm, s2_hbm, ei_hbm, zeros_hbm, hp_hbm,
               src_v, dst_v, ssrc_v, sdst_v, rows_v, stage_v, h_sh, sem):
    _edge_body(z_hbm, s2_hbm, ei_hbm, zeros_hbm, hp_hbm,
               src_v, dst_v, ssrc_v, sdst_v, rows_v, stage_v, h_sh, sem)


def _fin_body(hp_ref, out_ref):
    a = hp_ref[0]
    b = hp_ref[1]
    den = a[:, D:D + 1] + b[:, D:D + 1]
    out_ref[...] = (a[:, :D] + b[:, :D]) / (den + 1e-16)


def _finalize(hp):
    return pl.pallas_call(
        _fin_body,
        grid=(N_NODES // ROW_BLK,),
        in_specs=[pl.BlockSpec((NC, ROW_BLK, DP), lambda i: (0, i, 0))],
        out_specs=pl.BlockSpec((ROW_BLK, D), lambda i: (i, 0)),
        out_shape=jax.ShapeDtypeStruct((N_NODES, D), jnp.float32),
    )(hp)


def kernel(x, edge_index, W_src, W_dst, attn_w):
    z, s2 = _project(x, W_src, W_dst, attn_w)
    zeros = jnp.zeros((N_NODES, DP), jnp.float32)
    hp = _edge_pass(z, s2, edge_index, zeros)
    return _finalize(hp)


# speedup vs baseline: 10.5627x; 10.5627x over previous
import functools

import jax
import jax.numpy as jnp
from jax import lax
from jax.experimental import pallas as pl
from jax.experimental.pallas import tpu as pltpu
from jax.experimental.pallas import tpu_sc as plsc

N_NODES = 10000
N_EDGES = 320000
D = 128
NC = 2
NS = 16
E_PER = N_EDGES // (NC * NS)
DEN_STRIDE = 10240
RB = 624
ROW_BLK = 1000


def _proj_body(x_ref, ws_ref, wd_ref, aw_ref, z_ref, s2_ref):
    xb = x_ref[...]
    zs = lax.dot_general(xb, ws_ref[...], (((1,), (1,)), ((), ())),
                         preferred_element_type=jnp.float32)
    zd = lax.dot_general(xb, wd_ref[...], (((1,), (1,)), ((), ())),
                         preferred_element_type=jnp.float32)
    z_ref[...] = zs
    a = aw_ref[...]
    s_src = lax.dot_general(a[:, :D], zs, (((1,), (1,)), ((), ())),
                            preferred_element_type=jnp.float32)
    s_dst = lax.dot_general(a[:, D:], zd, (((1,), (1,)), ((), ())),
                            preferred_element_type=jnp.float32)
    s2_ref[0] = jnp.concatenate([s_src, s_dst], axis=0)


def _project(x, W_src, W_dst, attn_w):
    return pl.pallas_call(
        _proj_body,
        grid=(N_NODES // ROW_BLK,),
        in_specs=[
            pl.BlockSpec((ROW_BLK, D), lambda i: (i, 0)),
            pl.BlockSpec((D, D), lambda i: (0, 0)),
            pl.BlockSpec((D, D), lambda i: (0, 0)),
            pl.BlockSpec((1, 2 * D), lambda i: (0, 0)),
        ],
        out_specs=[
            pl.BlockSpec((ROW_BLK, D), lambda i: (i, 0)),
            pl.BlockSpec((1, 2, ROW_BLK), lambda i: (i, 0, 0)),
        ],
        out_shape=[
            jax.ShapeDtypeStruct((N_NODES, D), jnp.float32),
            jax.ShapeDtypeStruct((N_NODES // ROW_BLK, 2, ROW_BLK),
                                 jnp.float32),
        ],
    )(x, W_src, W_dst, attn_w)


def _edge_body(z_hbm, ssrc_hbm, sdst_hbm, src_hbm, dst_hbm, z2_hbm, z1_hbm,
               hp_hbm, den_hbm,
               src_v, dst_v, ssrc_v, sdst_v, rows_v, stage_v, ex_v, den_v,
               h_sh, den_sh, sem):
    c = lax.axis_index("c")
    s = lax.axis_index("s")
    base = (c * NS + s) * E_PER
    pltpu.sync_copy(src_hbm.at[pl.ds(base, E_PER)], src_v)
    pltpu.sync_copy(dst_hbm.at[pl.ds(base, E_PER)], dst_v)
    pltpu.sync_copy(ssrc_hbm, ssrc_v)
    pltpu.sync_copy(sdst_hbm, sdst_v)
    rbase = s * RB
    pltpu.sync_copy(z2_hbm.at[pl.ds(rbase, RB)], h_sh.at[pl.ds(rbase, RB)])
    pltpu.sync_copy(z1_hbm.at[pl.ds(rbase, RB)], den_v)
    pltpu.sync_copy(den_v, den_sh.at[pl.ds(rbase, RB)])

    @pl.when(s == 0)
    def _zero_tail():
        pltpu.sync_copy(z2_hbm.at[pl.ds(NS * RB, 16)],
                        h_sh.at[pl.ds(NS * RB, 16)])
        pltpu.sync_copy(den_v.at[pl.ds(0, 16)],
                        den_sh.at[pl.ds(NS * RB, 16)])

    plsc.subcore_barrier()

    def body(g, carry):
        eb = g * 16
        srcv = src_v[pl.ds(eb, 16)]
        dstv = dst_v[pl.ds(eb, 16)]
        sv = plsc.load_gather(ssrc_v, [srcv])
        dv = plsc.load_gather(sdst_v, [dstv])
        ev = sv + dv
        ev = jnp.where(ev >= 0.0, ev, ev * 0.01)
        exv = jnp.exp(ev)
        ex_v[pl.ds(0, 16)] = exv
        ex_v[pl.ds(16, 16)] = exv
        pltpu.async_copy(z_hbm.at[srcv], rows_v, sem).wait()
        for j in range(16):
            exj = plsc.load_gather(ex_v, [jnp.full((16,), 16 + j, jnp.int32)])
            for cc in range(8):
                stage_v[j, pl.ds(cc * 16, 16)] = (
                    rows_v[j, pl.ds(cc * 16, 16)] * exj)
        pltpu.sync_copy(stage_v, h_sh.at[dstv], add=True)
        pltpu.sync_copy(ex_v.at[pl.ds(0, 16)], den_sh.at[dstv], add=True)
        return carry

    lax.fori_loop(0, E_PER // 16, body, 0)
    plsc.subcore_barrier()
    pltpu.sync_copy(h_sh.at[pl.ds(rbase, RB)],
                    hp_hbm.at[c, pl.ds(rbase, RB)])
    pltpu.sync_copy(den_sh.at[pl.ds(rbase, RB)], den_v)
    pltpu.sync_copy(den_v, den_hbm.at[pl.ds(c * DEN_STRIDE + rbase, RB)])

    @pl.when(s == 0)
    def _dump_tail():
        pltpu.sync_copy(h_sh.at[pl.ds(NS * RB, 16)],
                        hp_hbm.at[c, pl.ds(NS * RB, 16)])
        pltpu.sync_copy(den_sh.at[pl.ds(NS * RB, 16)],
                        den_v.at[pl.ds(0, 16)])
        pltpu.sync_copy(den_v.at[pl.ds(0, 16)],
                        den_hbm.at[pl.ds(c * DEN_STRIDE + NS * RB, 16)])


@functools.lru_cache(maxsize=1)
def _edge_pass_fn():
    return pl.kernel(
        _edge_body,
        out_type=(
            jax.ShapeDtypeStruct((NC, N_NODES, D), jnp.float32),
            jax.ShapeDtypeStruct((NC * DEN_STRIDE,), jnp.float32),
        ),
        mesh=plsc.VectorSubcoreMesh(core_axis_name="c", subcore_axis_name="s"),
        compiler_params=pltpu.CompilerParams(needs_layout_passes=False),
        scratch_types=[
            pltpu.VMEM((E_PER,), jnp.int32),
            pltpu.VMEM((E_PER,), jnp.int32),
            pltpu.VMEM((N_NODES,), jnp.float32),
            pltpu.VMEM((N_NODES,), jnp.float32),
            pltpu.VMEM((16, D), jnp.float32),
            pltpu.VMEM((16, D), jnp.float32),
            pltpu.VMEM((32,), jnp.float32),
            pltpu.VMEM((RB,), jnp.float32),
            pltpu.VMEM_SHARED((N_NODES, D), jnp.float32),
            pltpu.VMEM_SHARED((N_NODES,), jnp.float32),
            pltpu.SemaphoreType.DMA,
        ],
    )


def _fin_body(hp_hbm, den_hbm, out_hbm, a_v, b_v, o_v, d0_v, d1_v, r_v):
    c = lax.axis_index("c")
    s = lax.axis_index("s")
    wid = c * NS + s
    n_groups = N_NODES // 16
    per_w = 20

    for k in range(per_w):
        g = wid * per_w + k

        @pl.when(g < n_groups)
        def _do():
            rb = g * 16
            pltpu.sync_copy(hp_hbm.at[0, pl.ds(rb, 16)], a_v)
            pltpu.sync_copy(hp_hbm.at[1, pl.ds(rb, 16)], b_v)
            pltpu.sync_copy(den_hbm.at[pl.ds(rb, 16)], d0_v)
            pltpu.sync_copy(den_hbm.at[pl.ds(DEN_STRIDE + rb, 16)], d1_v)
            recip = 1.0 / (d0_v[...] + d1_v[...] + 1e-16)
            r_v[pl.ds(0, 16)] = recip
            r_v[pl.ds(16, 16)] = recip
            for j in range(16):
                rj = plsc.load_gather(r_v, [jnp.full((16,), 16 + j, jnp.int32)])
                for cc in range(8):
                    o_v[j, pl.ds(cc * 16, 16)] = (
                        a_v[j, pl.ds(cc * 16, 16)]
                        + b_v[j, pl.ds(cc * 16, 16)]) * rj
            pltpu.sync_copy(o_v, out_hbm.at[pl.ds(rb, 16)])


@functools.lru_cache(maxsize=1)
def _fin_pass_fn():
    return pl.kernel(
        _fin_body,
        out_type=jax.ShapeDtypeStruct((N_NODES, D), jnp.float32),
        mesh=plsc.VectorSubcoreMesh(core_axis_name="c", subcore_axis_name="s"),
        compiler_params=pltpu.CompilerParams(needs_layout_passes=False),
        scratch_types=[
            pltpu.VMEM((16, D), jnp.float32),
            pltpu.VMEM((16, D), jnp.float32),
            pltpu.VMEM((16, D), jnp.float32),
            pltpu.VMEM((16,), jnp.float32),
            pltpu.VMEM((16,), jnp.float32),
            pltpu.VMEM((32,), jnp.float32),
        ],
    )


def kernel(x, edge_index, W_src, W_dst, attn_w):
    z, s3 = _project(x, W_src, W_dst, attn_w)
    ssrc = s3[:, 0, :].reshape(N_NODES)
    sdst = s3[:, 1, :].reshape(N_NODES)
    src = edge_index[0]
    dst = edge_index[1]
    zeros2 = jnp.zeros((N_NODES, D), jnp.float32)
    zeros1 = jnp.zeros((N_NODES,), jnp.float32)
    hp, den = _edge_pass_fn()(z, ssrc, sdst, src, dst, zeros2, zeros1)
    return _fin_pass_fn()(hp, den)


# speedup vs baseline: 19.9726x; 1.8909x over previous
import functools

import jax
import jax.numpy as jnp
from jax import lax
from jax.experimental import pallas as pl
from jax.experimental.pallas import tpu as pltpu
from jax.experimental.pallas import tpu_sc as plsc

N_NODES = 10000
N_EDGES = 320000
D = 128
NC = 2
NS = 16
E_PER = N_EDGES // (NC * NS)
DEN_STRIDE = 10240
RB = 624
ROW_BLK = 1000


def _proj_body(x_ref, ws_ref, wd_ref, aw_ref, z_ref, s2_ref):
    xb = x_ref[...]
    zs = lax.dot_general(xb, ws_ref[...], (((1,), (1,)), ((), ())),
                         preferred_element_type=jnp.float32)
    zd = lax.dot_general(xb, wd_ref[...], (((1,), (1,)), ((), ())),
                         preferred_element_type=jnp.float32)
    z_ref[...] = zs
    a = aw_ref[...]
    s_src = lax.dot_general(a[:, :D], zs, (((1,), (1,)), ((), ())),
                            preferred_element_type=jnp.float32)
    s_dst = lax.dot_general(a[:, D:], zd, (((1,), (1,)), ((), ())),
                            preferred_element_type=jnp.float32)
    s2_ref[0] = jnp.concatenate([s_src, s_dst], axis=0)


def _project(x, W_src, W_dst, attn_w):
    return pl.pallas_call(
        _proj_body,
        grid=(N_NODES // ROW_BLK,),
        in_specs=[
            pl.BlockSpec((ROW_BLK, D), lambda i: (i, 0)),
            pl.BlockSpec((D, D), lambda i: (0, 0)),
            pl.BlockSpec((D, D), lambda i: (0, 0)),
            pl.BlockSpec((1, 2 * D), lambda i: (0, 0)),
        ],
        out_specs=[
            pl.BlockSpec((ROW_BLK, D), lambda i: (i, 0)),
            pl.BlockSpec((1, 2, ROW_BLK), lambda i: (i, 0, 0)),
        ],
        out_shape=[
            jax.ShapeDtypeStruct((N_NODES, D), jnp.float32),
            jax.ShapeDtypeStruct((N_NODES // ROW_BLK, 2, ROW_BLK),
                                 jnp.float32),
        ],
    )(x, W_src, W_dst, attn_w)


G = 16
NG = 624
TAIL = E_PER - NG * G


def _edge_body(z_hbm, ssrc_hbm, sdst_hbm, src_hbm, dst_hbm, z2_hbm, z1_hbm,
               hp_hbm, den_hbm,
               src_v, dst_v, ssrc_v, sdst_v,
               rows0, rows1, stage0, stage1, ex0, ex1, dsti0, dsti1, den_v,
               h_sh, den_sh, gsem0, gsem1, hsem0, hsem1, dsem0, dsem1):
    c = lax.axis_index("c")
    s = lax.axis_index("s")
    base = (c * NS + s) * E_PER
    pltpu.sync_copy(src_hbm.at[pl.ds(base, E_PER)], src_v)
    pltpu.sync_copy(dst_hbm.at[pl.ds(base, E_PER)], dst_v)
    pltpu.sync_copy(ssrc_hbm, ssrc_v)
    pltpu.sync_copy(sdst_hbm, sdst_v)
    rbase = s * RB
    pltpu.sync_copy(z2_hbm.at[pl.ds(rbase, RB)], h_sh.at[pl.ds(rbase, RB)])
    pltpu.sync_copy(z1_hbm.at[pl.ds(rbase, RB)], den_v)
    pltpu.sync_copy(den_v, den_sh.at[pl.ds(rbase, RB)])

    @pl.when(s == 0)
    def _zero_tail():
        pltpu.sync_copy(z2_hbm.at[pl.ds(NS * RB, 16)],
                        h_sh.at[pl.ds(NS * RB, 16)])
        pltpu.sync_copy(den_v.at[pl.ds(0, 16)],
                        den_sh.at[pl.ds(NS * RB, 16)])

    plsc.subcore_barrier()

    rowsb = (rows0, rows1)
    stageb = (stage0, stage1)
    exb = (ex0, ex1)
    dstib = (dsti0, dsti1)
    gsemb = (gsem0, gsem1)
    hsemb = (hsem0, hsem1)
    dsemb = (dsem0, dsem1)

    def issue_gather(g, b):
        pltpu.async_copy(z_hbm.at[src_v.at[pl.ds(g * G, G)]],
                         rowsb[b], gsemb[b])

    def wait_gather(b):
        pltpu.make_async_copy(z_hbm.at[src_v.at[pl.ds(0, G)]],
                              rowsb[b], gsemb[b]).wait()

    def compute_ex(g, b):
        eb = g * G
        for q in range(G // 16):
            s16 = src_v[pl.ds(eb + q * 16, 16)]
            d16 = dst_v[pl.ds(eb + q * 16, 16)]
            dstib[b][pl.ds(q * 16, 16)] = d16
            sv = plsc.load_gather(ssrc_v, [s16])
            dv = plsc.load_gather(sdst_v, [d16])
            ev = sv + dv
            ev = jnp.where(ev >= 0.0, ev, ev * 0.01)
            exb[b][pl.ds(16 + q * 16, 16)] = jnp.exp(ev)

    def scale(b):
        for j in range(G):
            exj = plsc.load_gather(
                exb[b], [jnp.full((16,), 16 + j, jnp.int32)])
            for cc in range(8):
                stageb[b][j, pl.ds(cc * 16, 16)] = (
                    rowsb[b][j, pl.ds(cc * 16, 16)] * exj)

    def issue_scatters(b):
        pltpu.async_copy(stageb[b], h_sh.at[dstib[b]], hsemb[b], add=True)
        pltpu.async_copy(exb[b].at[pl.ds(16, G)], den_sh.at[dstib[b]],
                         dsemb[b], add=True)

    def wait_scatters(b):
        pltpu.make_async_copy(stageb[b], h_sh.at[dstib[b]], hsemb[b]).wait()
        pltpu.make_async_copy(exb[b].at[pl.ds(16, G)], den_sh.at[dstib[b]],
                              dsemb[b]).wait()

    for b in range(2):
        issue_gather(b, b)
    for b in range(2):
        wait_gather(b)
        compute_ex(b, b)
        scale(b)
        issue_scatters(b)
        issue_gather(b + 2, b)

    def cyc(i, carry):
        for b in range(2):
            g = 2 * i + b
            wait_gather(b)
            wait_scatters(b)
            compute_ex(g, b)
            scale(b)
            issue_scatters(b)
            gp = jnp.minimum(g + 2, NG - 1)
            issue_gather(gp, b)
        return carry

    lax.fori_loop(1, NG // 2, cyc, 0)

    for b in range(2):
        wait_gather(b)
        wait_scatters(b)

    eb = NG * G
    srcv = src_v[pl.ds(eb, 16)]
    dstv = dst_v[pl.ds(eb, 16)]
    sv = plsc.load_gather(ssrc_v, [srcv])
    dv = plsc.load_gather(sdst_v, [dstv])
    ev = sv + dv
    ev = jnp.where(ev >= 0.0, ev, ev * 0.01)
    ex0[pl.ds(16, 16)] = jnp.exp(ev)
    pltpu.async_copy(z_hbm.at[srcv], rows0.at[pl.ds(0, 16)], gsem0).wait()
    for j in range(16):
        exj = plsc.load_gather(ex0, [jnp.full((16,), 16 + j, jnp.int32)])
        for cc in range(8):
            stage0[j, pl.ds(cc * 16, 16)] = (
                rows0[j, pl.ds(cc * 16, 16)] * exj)
    pltpu.sync_copy(stage0.at[pl.ds(0, 16)], h_sh.at[dstv], add=True)
    pltpu.sync_copy(ex0.at[pl.ds(16, 16)], den_sh.at[dstv], add=True)

    plsc.subcore_barrier()
    pltpu.sync_copy(h_sh.at[pl.ds(rbase, RB)],
                    hp_hbm.at[c, pl.ds(rbase, RB)])
    pltpu.sync_copy(den_sh.at[pl.ds(rbase, RB)], den_v)
    pltpu.sync_copy(den_v, den_hbm.at[pl.ds(c * DEN_STRIDE + rbase, RB)])

    @pl.when(s == 0)
    def _dump_tail():
        pltpu.sync_copy(h_sh.at[pl.ds(NS * RB, 16)],
                        hp_hbm.at[c, pl.ds(NS * RB, 16)])
        pltpu.sync_copy(den_sh.at[pl.ds(NS * RB, 16)],
                        den_v.at[pl.ds(0, 16)])
        pltpu.sync_copy(den_v.at[pl.ds(0, 16)],
                        den_hbm.at[pl.ds(c * DEN_STRIDE + NS * RB, 16)])


@functools.lru_cache(maxsize=1)
def _edge_pass_fn():
    return pl.kernel(
        _edge_body,
        out_type=(
            jax.ShapeDtypeStruct((NC, N_NODES, D), jnp.float32),
            jax.ShapeDtypeStruct((NC * DEN_STRIDE,), jnp.float32),
        ),
        mesh=plsc.VectorSubcoreMesh(core_axis_name="c", subcore_axis_name="s"),
        compiler_params=pltpu.CompilerParams(needs_layout_passes=False),
        scratch_types=[
            pltpu.VMEM((E_PER,), jnp.int32),
            pltpu.VMEM((E_PER,), jnp.int32),
            pltpu.VMEM((N_NODES,), jnp.float32),
            pltpu.VMEM((N_NODES,), jnp.float32),
            pltpu.VMEM((G, D), jnp.float32),
            pltpu.VMEM((G, D), jnp.float32),
            pltpu.VMEM((G, D), jnp.float32),
            pltpu.VMEM((G, D), jnp.float32),
            pltpu.VMEM((16 + G,), jnp.float32),
            pltpu.VMEM((16 + G,), jnp.float32),
            pltpu.VMEM((G,), jnp.int32),
            pltpu.VMEM((G,), jnp.int32),
            pltpu.VMEM((RB,), jnp.float32),
            pltpu.VMEM_SHARED((N_NODES, D), jnp.float32),
            pltpu.VMEM_SHARED((N_NODES,), jnp.float32),
            pltpu.SemaphoreType.DMA,
            pltpu.SemaphoreType.DMA,
            pltpu.SemaphoreType.DMA,
            pltpu.SemaphoreType.DMA,
            pltpu.SemaphoreType.DMA,
            pltpu.SemaphoreType.DMA,
        ],
    )


def _fin_body(hp_hbm, den_hbm, out_hbm, a_v, b_v, o_v, d0_v, d1_v, r_v):
    c = lax.axis_index("c")
    s = lax.axis_index("s")
    wid = c * NS + s
    n_groups = N_NODES // 16
    per_w = 20

    for k in range(per_w):
        g = wid * per_w + k

        @pl.when(g < n_groups)
        def _do():
            rb = g * 16
            pltpu.sync_copy(hp_hbm.at[0, pl.ds(rb, 16)], a_v)
            pltpu.sync_copy(hp_hbm.at[1, pl.ds(rb, 16)], b_v)
            pltpu.sync_copy(den_hbm.at[pl.ds(rb, 16)], d0_v)
            pltpu.sync_copy(den_hbm.at[pl.ds(DEN_STRIDE + rb, 16)], d1_v)
            recip = 1.0 / (d0_v[...] + d1_v[...] + 1e-16)
            r_v[pl.ds(0, 16)] = recip
            r_v[pl.ds(16, 16)] = recip
            for j in range(16):
                rj = plsc.load_gather(r_v, [jnp.full((16,), 16 + j, jnp.int32)])
                for cc in range(8):
                    o_v[j, pl.ds(cc * 16, 16)] = (
                        a_v[j, pl.ds(cc * 16, 16)]
                        + b_v[j, pl.ds(cc * 16, 16)]) * rj
            pltpu.sync_copy(o_v, out_hbm.at[pl.ds(rb, 16)])


@functools.lru_cache(maxsize=1)
def _fin_pass_fn():
    return pl.kernel(
        _fin_body,
        out_type=jax.ShapeDtypeStruct((N_NODES, D), jnp.float32),
        mesh=plsc.VectorSubcoreMesh(core_axis_name="c", subcore_axis_name="s"),
        compiler_params=pltpu.CompilerParams(needs_layout_passes=False),
        scratch_types=[
            pltpu.VMEM((16, D), jnp.float32),
            pltpu.VMEM((16, D), jnp.float32),
            pltpu.VMEM((16, D), jnp.float32),
            pltpu.VMEM((16,), jnp.float32),
            pltpu.VMEM((16,), jnp.float32),
            pltpu.VMEM((32,), jnp.float32),
        ],
    )


def kernel(x, edge_index, W_src, W_dst, attn_w):
    z, s3 = _project(x, W_src, W_dst, attn_w)
    ssrc = s3[:, 0, :].reshape(N_NODES)
    sdst = s3[:, 1, :].reshape(N_NODES)
    src = edge_index[0]
    dst = edge_index[1]
    zeros2 = jnp.zeros((N_NODES, D), jnp.float32)
    zeros1 = jnp.zeros((N_NODES,), jnp.float32)
    hp, den = _edge_pass_fn()(z, ssrc, sdst, src, dst, zeros2, zeros1)
    return _fin_pass_fn()(hp, den)


# speedup vs baseline: 25.1574x; 1.2596x over previous
import functools

import jax
import jax.numpy as jnp
from jax import lax
from jax.experimental import pallas as pl
from jax.experimental.pallas import tpu as pltpu
from jax.experimental.pallas import tpu_sc as plsc

N_NODES = 10000
N_EDGES = 320000
D = 128
NC = 2
NS = 16
E_PER = N_EDGES // (NC * NS)
DEN_STRIDE = 10240
RB = 624
ROW_BLK = 1000


def _proj_body(x_ref, ws_ref, wd_ref, aw_ref, z_ref, s2_ref):
    xb = x_ref[...]
    zs = lax.dot_general(xb, ws_ref[...], (((1,), (1,)), ((), ())),
                         preferred_element_type=jnp.float32)
    zd = lax.dot_general(xb, wd_ref[...], (((1,), (1,)), ((), ())),
                         preferred_element_type=jnp.float32)
    z_ref[...] = zs
    a = aw_ref[...]
    s_src = lax.dot_general(a[:, :D], zs, (((1,), (1,)), ((), ())),
                            preferred_element_type=jnp.float32)
    s_dst = lax.dot_general(a[:, D:], zd, (((1,), (1,)), ((), ())),
                            preferred_element_type=jnp.float32)
    s2_ref[0] = jnp.concatenate([s_src, s_dst], axis=0)


def _project(x, W_src, W_dst, attn_w):
    return pl.pallas_call(
        _proj_body,
        grid=(N_NODES // ROW_BLK,),
        in_specs=[
            pl.BlockSpec((ROW_BLK, D), lambda i: (i, 0)),
            pl.BlockSpec((D, D), lambda i: (0, 0)),
            pl.BlockSpec((D, D), lambda i: (0, 0)),
            pl.BlockSpec((1, 2 * D), lambda i: (0, 0)),
        ],
        out_specs=[
            pl.BlockSpec((ROW_BLK, D), lambda i: (i, 0)),
            pl.BlockSpec((1, 2, ROW_BLK), lambda i: (i, 0, 0)),
        ],
        out_shape=[
            jax.ShapeDtypeStruct((N_NODES, D), jnp.float32),
            jax.ShapeDtypeStruct((N_NODES // ROW_BLK, 2, ROW_BLK),
                                 jnp.float32),
        ],
    )(x, W_src, W_dst, attn_w)


G = 32
NGH = 156
EH = NGH * G
NH = 2


def _edge_body(z_hbm, ssrc_hbm, sdst_hbm, src_hbm, dst_hbm, z2_hbm, z1_hbm,
               hp_hbm, den_hbm,
               src_v, dst_v, ssrc_v, sdst_v,
               rows0, rows1, stage0, stage1, ex0, ex1, dsti0, dsti1, den_v,
               h_sh, den_sh, gsem0, gsem1, hsem0, hsem1, dsem0, dsem1):
    c = lax.axis_index("c")
    s = lax.axis_index("s")
    base = (c * NS + s) * E_PER
    pltpu.sync_copy(ssrc_hbm, ssrc_v)
    pltpu.sync_copy(sdst_hbm, sdst_v)
    rbase = s * RB
    pltpu.sync_copy(z2_hbm.at[pl.ds(rbase, RB)], h_sh.at[pl.ds(rbase, RB)])
    pltpu.sync_copy(z1_hbm.at[pl.ds(rbase, RB)], den_v)
    pltpu.sync_copy(den_v, den_sh.at[pl.ds(rbase, RB)])

    @pl.when(s == 0)
    def _zero_tail():
        pltpu.sync_copy(z2_hbm.at[pl.ds(NS * RB, 16)],
                        h_sh.at[pl.ds(NS * RB, 16)])
        pltpu.sync_copy(den_v.at[pl.ds(0, 16)],
                        den_sh.at[pl.ds(NS * RB, 16)])

    plsc.subcore_barrier()

    rowsb = (rows0, rows1)
    stageb = (stage0, stage1)
    exb = (ex0, ex1)
    dstib = (dsti0, dsti1)
    gsemb = (gsem0, gsem1)
    hsemb = (hsem0, hsem1)
    dsemb = (dsem0, dsem1)

    def issue_gather(g, b):
        pltpu.async_copy(z_hbm.at[src_v.at[pl.ds(g * G, G)]],
                         rowsb[b], gsemb[b])

    def wait_gather(b):
        pltpu.make_async_copy(z_hbm.at[src_v.at[pl.ds(0, G)]],
                              rowsb[b], gsemb[b]).wait()

    def compute_ex(g, b):
        eb = g * G
        for q in range(G // 16):
            s16 = src_v[pl.ds(eb + q * 16, 16)]
            d16 = dst_v[pl.ds(eb + q * 16, 16)]
            dstib[b][pl.ds(q * 16, 16)] = d16
            sv = plsc.load_gather(ssrc_v, [s16])
            dv = plsc.load_gather(sdst_v, [d16])
            ev = sv + dv
            ev = jnp.where(ev >= 0.0, ev, ev * 0.01)
            exb[b][pl.ds(16 + q * 16, 16)] = jnp.exp(ev)

    def scale(b):
        for j in range(G):
            exj = plsc.load_gather(
                exb[b], [jnp.full((16,), 16 + j, jnp.int32)])
            for cc in range(8):
                stageb[b][j, pl.ds(cc * 16, 16)] = (
                    rowsb[b][j, pl.ds(cc * 16, 16)] * exj)

    def issue_scatters(b):
        pltpu.async_copy(stageb[b], h_sh.at[dstib[b]], hsemb[b], add=True)
        pltpu.async_copy(exb[b].at[pl.ds(16, G)], den_sh.at[dstib[b]],
                         dsemb[b], add=True)

    def wait_scatters(b):
        pltpu.make_async_copy(stageb[b], h_sh.at[dstib[b]], hsemb[b]).wait()
        pltpu.make_async_copy(exb[b].at[pl.ds(16, G)], den_sh.at[dstib[b]],
                              dsemb[b]).wait()

    def half(h, carry):
        ebase = base + h * EH
        pltpu.sync_copy(src_hbm.at[pl.ds(ebase, EH)], src_v)
        pltpu.sync_copy(dst_hbm.at[pl.ds(ebase, EH)], dst_v)
        for b in range(2):
            issue_gather(b, b)

        def cyc(i, carry2):
            for b in range(2):
                g = 2 * i + b
                wait_gather(b)

                @pl.when(i > 0)
                def _w():
                    wait_scatters(b)

                compute_ex(g, b)
                scale(b)
                issue_scatters(b)
                gp = jnp.minimum(g + 2, NGH - 1)
                issue_gather(gp, b)
            return carry2

        lax.fori_loop(0, NGH // 2, cyc, 0)
        for b in range(2):
            wait_gather(b)
            wait_scatters(b)
        return carry

    lax.fori_loop(0, NH, half, 0)

    pltpu.sync_copy(src_hbm.at[pl.ds(base + NH * EH, 16)],
                    src_v.at[pl.ds(0, 16)])
    pltpu.sync_copy(dst_hbm.at[pl.ds(base + NH * EH, 16)],
                    dst_v.at[pl.ds(0, 16)])
    srcv = src_v[pl.ds(0, 16)]
    dstv = dst_v[pl.ds(0, 16)]
    sv = plsc.load_gather(ssrc_v, [srcv])
    dv = plsc.load_gather(sdst_v, [dstv])
    ev = sv + dv
    ev = jnp.where(ev >= 0.0, ev, ev * 0.01)
    ex0[pl.ds(16, 16)] = jnp.exp(ev)
    pltpu.async_copy(z_hbm.at[srcv], rows0.at[pl.ds(0, 16)], gsem0).wait()
    for j in range(16):
        exj = plsc.load_gather(ex0, [jnp.full((16,), 16 + j, jnp.int32)])
        for cc in range(8):
            stage0[j, pl.ds(cc * 16, 16)] = (
                rows0[j, pl.ds(cc * 16, 16)] * exj)
    pltpu.sync_copy(stage0.at[pl.ds(0, 16)], h_sh.at[dstv], add=True)
    pltpu.sync_copy(ex0.at[pl.ds(16, 16)], den_sh.at[dstv], add=True)

    plsc.subcore_barrier()
    pltpu.sync_copy(h_sh.at[pl.ds(rbase, RB)],
                    hp_hbm.at[c, pl.ds(rbase, RB)])
    pltpu.sync_copy(den_sh.at[pl.ds(rbase, RB)], den_v)
    pltpu.sync_copy(den_v, den_hbm.at[pl.ds(c * DEN_STRIDE + rbase, RB)])

    @pl.when(s == 0)
    def _dump_tail():
        pltpu.sync_copy(h_sh.at[pl.ds(NS * RB, 16)],
                        hp_hbm.at[c, pl.ds(NS * RB, 16)])
        pltpu.sync_copy(den_sh.at[pl.ds(NS * RB, 16)],
                        den_v.at[pl.ds(0, 16)])
        pltpu.sync_copy(den_v.at[pl.ds(0, 16)],
                        den_hbm.at[pl.ds(c * DEN_STRIDE + NS * RB, 16)])


@functools.lru_cache(maxsize=1)
def _edge_pass_fn():
    return pl.kernel(
        _edge_body,
        out_type=(
            jax.ShapeDtypeStruct((NC, N_NODES, D), jnp.float32),
            jax.ShapeDtypeStruct((NC * DEN_STRIDE,), jnp.float32),
        ),
        mesh=plsc.VectorSubcoreMesh(core_axis_name="c", subcore_axis_name="s"),
        compiler_params=pltpu.CompilerParams(needs_layout_passes=False),
        scratch_types=[
            pltpu.VMEM((EH,), jnp.int32),
            pltpu.VMEM((EH,), jnp.int32),
            pltpu.VMEM((N_NODES,), jnp.float32),
            pltpu.VMEM((N_NODES,), jnp.float32),
            pltpu.VMEM((G, D), jnp.float32),
            pltpu.VMEM((G, D), jnp.float32),
            pltpu.VMEM((G, D), jnp.float32),
            pltpu.VMEM((G, D), jnp.float32),
            pltpu.VMEM((16 + G,), jnp.float32),
            pltpu.VMEM((16 + G,), jnp.float32),
            pltpu.VMEM((G,), jnp.int32),
            pltpu.VMEM((G,), jnp.int32),
            pltpu.VMEM((RB,), jnp.float32),
            pltpu.VMEM_SHARED((N_NODES, D), jnp.float32),
            pltpu.VMEM_SHARED((N_NODES,), jnp.float32),
            pltpu.SemaphoreType.DMA,
            pltpu.SemaphoreType.DMA,
            pltpu.SemaphoreType.DMA,
            pltpu.SemaphoreType.DMA,
            pltpu.SemaphoreType.DMA,
            pltpu.SemaphoreType.DMA,
        ],
    )


def _fin_body(hp_hbm, den_hbm, out_hbm, a_v, b_v, o_v, d0_v, d1_v, r_v):
    c = lax.axis_index("c")
    s = lax.axis_index("s")
    wid = c * NS + s
    n_groups = N_NODES // 16
    per_w = 20

    for k in range(per_w):
        g = wid * per_w + k

        @pl.when(g < n_groups)
        def _do():
            rb = g * 16
            pltpu.sync_copy(hp_hbm.at[0, pl.ds(rb, 16)], a_v)
            pltpu.sync_copy(hp_hbm.at[1, pl.ds(rb, 16)], b_v)
            pltpu.sync_copy(den_hbm.at[pl.ds(rb, 16)], d0_v)
            pltpu.sync_copy(den_hbm.at[pl.ds(DEN_STRIDE + rb, 16)], d1_v)
            recip = 1.0 / (d0_v[...] + d1_v[...] + 1e-16)
            r_v[pl.ds(0, 16)] = recip
            r_v[pl.ds(16, 16)] = recip
            for j in range(16):
                rj = plsc.load_gather(r_v, [jnp.full((16,), 16 + j, jnp.int32)])
                for cc in range(8):
                    o_v[j, pl.ds(cc * 16, 16)] = (
                        a_v[j, pl.ds(cc * 16, 16)]
                        + b_v[j, pl.ds(cc * 16, 16)]) * rj
            pltpu.sync_copy(o_v, out_hbm.at[pl.ds(rb, 16)])


@functools.lru_cache(maxsize=1)
def _fin_pass_fn():
    return pl.kernel(
        _fin_body,
        out_type=jax.ShapeDtypeStruct((N_NODES, D), jnp.float32),
        mesh=plsc.VectorSubcoreMesh(core_axis_name="c", subcore_axis_name="s"),
        compiler_params=pltpu.CompilerParams(needs_layout_passes=False),
        scratch_types=[
            pltpu.VMEM((16, D), jnp.float32),
            pltpu.VMEM((16, D), jnp.float32),
            pltpu.VMEM((16, D), jnp.float32),
            pltpu.VMEM((16,), jnp.float32),
            pltpu.VMEM((16,), jnp.float32),
            pltpu.VMEM((32,), jnp.float32),
        ],
    )


def kernel(x, edge_index, W_src, W_dst, attn_w):
    z, s3 = _project(x, W_src, W_dst, attn_w)
    ssrc = s3[:, 0, :].reshape(N_NODES)
    sdst = s3[:, 1, :].reshape(N_NODES)
    src = edge_index[0]
    dst = edge_index[1]
    zeros2 = jnp.zeros((N_NODES, D), jnp.float32)
    zeros1 = jnp.zeros((N_NODES,), jnp.float32)
    hp, den = _edge_pass_fn()(z, ssrc, sdst, src, dst, zeros2, zeros1)
    return _fin_pass_fn()(hp, den)


# speedup vs baseline: 29.0057x; 1.1530x over previous
import functools

import jax
import jax.numpy as jnp
from jax import lax
from jax.experimental import pallas as pl
from jax.experimental.pallas import tpu as pltpu
from jax.experimental.pallas import tpu_sc as plsc

N_NODES = 10000
N_EDGES = 320000
D = 128
NC = 2
NS = 16
E_PER = N_EDGES // (NC * NS)
DEN_STRIDE = 10240
RB = 624
ROW_BLK = 1000


def _proj_body(x_ref, ws_ref, wd_ref, aw_ref, z_ref, s2_ref):
    xb = x_ref[...]
    zs = lax.dot_general(xb, ws_ref[...], (((1,), (1,)), ((), ())),
                         preferred_element_type=jnp.float32)
    zd = lax.dot_general(xb, wd_ref[...], (((1,), (1,)), ((), ())),
                         preferred_element_type=jnp.float32)
    z_ref[...] = zs
    a = aw_ref[...]
    s_src = lax.dot_general(a[:, :D], zs, (((1,), (1,)), ((), ())),
                            preferred_element_type=jnp.float32)
    s_dst = lax.dot_general(a[:, D:], zd, (((1,), (1,)), ((), ())),
                            preferred_element_type=jnp.float32)
    s2_ref[0] = jnp.concatenate([s_src, s_dst], axis=0)


def _project(x, W_src, W_dst, attn_w):
    return pl.pallas_call(
        _proj_body,
        grid=(N_NODES // ROW_BLK,),
        in_specs=[
            pl.BlockSpec((ROW_BLK, D), lambda i: (i, 0)),
            pl.BlockSpec((D, D), lambda i: (0, 0)),
            pl.BlockSpec((D, D), lambda i: (0, 0)),
            pl.BlockSpec((1, 2 * D), lambda i: (0, 0)),
        ],
        out_specs=[
            pl.BlockSpec((ROW_BLK, D), lambda i: (i, 0)),
            pl.BlockSpec((1, 2, ROW_BLK), lambda i: (i, 0, 0)),
        ],
        out_shape=[
            jax.ShapeDtypeStruct((N_NODES, D), jnp.float32),
            jax.ShapeDtypeStruct((N_NODES // ROW_BLK, 2, ROW_BLK),
                                 jnp.float32),
        ],
    )(x, W_src, W_dst, attn_w)


G = 32
NGH = 156
EH = NGH * G
NH = 2


def _edge_body(z_hbm, ssrc_hbm, sdst_hbm, src_hbm, dst_hbm,
               hp_hbm, den_hbm,
               src_v, dst_v, ssrc_v, sdst_v,
               rows0, rows1, stage0, stage1, ex0, ex1, dsti0, dsti1, den_v,
               h_sh, den_sh, gsem0, gsem1, hsem0, hsem1, dsem0, dsem1):
    c = lax.axis_index("c")
    s = lax.axis_index("s")
    base = (c * NS + s) * E_PER
    pltpu.sync_copy(ssrc_hbm, ssrc_v)
    pltpu.sync_copy(sdst_hbm, sdst_v)
    zero16 = jnp.zeros((16,), jnp.float32)
    for j in range(G):
        for cc in range(8):
            stage0[j, pl.ds(cc * 16, 16)] = zero16
    for t in range(RB // 16):
        den_v[pl.ds(t * 16, 16)] = zero16
    rbase = s * RB
    for t in range(RB // G):
        pltpu.sync_copy(stage0, h_sh.at[pl.ds(rbase + t * G, G)])
    pltpu.sync_copy(stage0.at[pl.ds(0, 16)],
                    h_sh.at[pl.ds(rbase + (RB // G) * G, 16)])
    pltpu.sync_copy(den_v, den_sh.at[pl.ds(rbase, RB)])

    @pl.when(s == 0)
    def _zero_tail():
        pltpu.sync_copy(stage0.at[pl.ds(0, 16)],
                        h_sh.at[pl.ds(NS * RB, 16)])
        pltpu.sync_copy(den_v.at[pl.ds(0, 16)],
                        den_sh.at[pl.ds(NS * RB, 16)])

    plsc.subcore_barrier()

    rowsb = (rows0, rows1)
    stageb = (stage0, stage1)
    exb = (ex0, ex1)
    dstib = (dsti0, dsti1)
    gsemb = (gsem0, gsem1)
    hsemb = (hsem0, hsem1)
    dsemb = (dsem0, dsem1)

    def issue_gather(g, b):
        pltpu.async_copy(z_hbm.at[src_v.at[pl.ds(g * G, G)]],
                         rowsb[b], gsemb[b])

    def wait_gather(b):
        pltpu.make_async_copy(z_hbm.at[src_v.at[pl.ds(0, G)]],
                              rowsb[b], gsemb[b]).wait()

    def compute_ex(g, b):
        eb = g * G
        for q in range(G // 16):
            s16 = src_v[pl.ds(eb + q * 16, 16)]
            d16 = dst_v[pl.ds(eb + q * 16, 16)]
            dstib[b][pl.ds(q * 16, 16)] = d16
            sv = plsc.load_gather(ssrc_v, [s16])
            dv = plsc.load_gather(sdst_v, [d16])
            ev = sv + dv
            ev = jnp.where(ev >= 0.0, ev, ev * 0.01)
            exb[b][pl.ds(16 + q * 16, 16)] = jnp.exp(ev)

    def scale(b):
        for j in range(G):
            exj = plsc.load_gather(
                exb[b], [jnp.full((16,), 16 + j, jnp.int32)])
            for cc in range(8):
                stageb[b][j, pl.ds(cc * 16, 16)] = (
                    rowsb[b][j, pl.ds(cc * 16, 16)] * exj)

    def issue_scatters(b):
        pltpu.async_copy(stageb[b], h_sh.at[dstib[b]], hsemb[b], add=True)
        pltpu.async_copy(exb[b].at[pl.ds(16, G)], den_sh.at[dstib[b]],
                         dsemb[b], add=True)

    def wait_scatters(b):
        pltpu.make_async_copy(stageb[b], h_sh.at[dstib[b]], hsemb[b]).wait()
        pltpu.make_async_copy(exb[b].at[pl.ds(16, G)], den_sh.at[dstib[b]],
                              dsemb[b]).wait()

    def half(h, carry):
        ebase = base + h * EH
        pltpu.sync_copy(src_hbm.at[pl.ds(ebase, EH)], src_v)
        pltpu.sync_copy(dst_hbm.at[pl.ds(ebase, EH)], dst_v)
        for b in range(2):
            issue_gather(b, b)

        def cyc(i, carry2):
            for b in range(2):
                g = 2 * i + b
                wait_gather(b)

                @pl.when(i > 0)
                def _w():
                    wait_scatters(b)

                compute_ex(g, b)
                scale(b)
                issue_scatters(b)
                gp = jnp.minimum(g + 2, NGH - 1)
                issue_gather(gp, b)
            return carry2

        lax.fori_loop(0, NGH // 2, cyc, 0)
        for b in range(2):
            wait_gather(b)
            wait_scatters(b)
        return carry

    lax.fori_loop(0, NH, half, 0)

    pltpu.sync_copy(src_hbm.at[pl.ds(base + NH * EH, 16)],
                    src_v.at[pl.ds(0, 16)])
    pltpu.sync_copy(dst_hbm.at[pl.ds(base + NH * EH, 16)],
                    dst_v.at[pl.ds(0, 16)])
    srcv = src_v[pl.ds(0, 16)]
    dstv = dst_v[pl.ds(0, 16)]
    sv = plsc.load_gather(ssrc_v, [srcv])
    dv = plsc.load_gather(sdst_v, [dstv])
    ev = sv + dv
    ev = jnp.where(ev >= 0.0, ev, ev * 0.01)
    ex0[pl.ds(16, 16)] = jnp.exp(ev)
    pltpu.async_copy(z_hbm.at[srcv], rows0.at[pl.ds(0, 16)], gsem0).wait()
    for j in range(16):
        exj = plsc.load_gather(ex0, [jnp.full((16,), 16 + j, jnp.int32)])
        for cc in range(8):
            stage0[j, pl.ds(cc * 16, 16)] = (
                rows0[j, pl.ds(cc * 16, 16)] * exj)
    pltpu.sync_copy(stage0.at[pl.ds(0, 16)], h_sh.at[dstv], add=True)
    pltpu.sync_copy(ex0.at[pl.ds(16, 16)], den_sh.at[dstv], add=True)

    plsc.subcore_barrier()
    pltpu.sync_copy(h_sh.at[pl.ds(rbase, RB)],
                    hp_hbm.at[c, pl.ds(rbase, RB)])
    pltpu.sync_copy(den_sh.at[pl.ds(rbase, RB)], den_v)
    pltpu.sync_copy(den_v, den_hbm.at[pl.ds(c * DEN_STRIDE + rbase, RB)])

    @pl.when(s == 0)
    def _dump_tail():
        pltpu.sync_copy(h_sh.at[pl.ds(NS * RB, 16)],
                        hp_hbm.at[c, pl.ds(NS * RB, 16)])
        pltpu.sync_copy(den_sh.at[pl.ds(NS * RB, 16)],
                        den_v.at[pl.ds(0, 16)])
        pltpu.sync_copy(den_v.at[pl.ds(0, 16)],
                        den_hbm.at[pl.ds(c * DEN_STRIDE + NS * RB, 16)])


@functools.lru_cache(maxsize=1)
def _edge_pass_fn():
    return pl.kernel(
        _edge_body,
        out_type=(
            jax.ShapeDtypeStruct((NC, N_NODES, D), jnp.float32),
            jax.ShapeDtypeStruct((NC * DEN_STRIDE,), jnp.float32),
        ),
        mesh=plsc.VectorSubcoreMesh(core_axis_name="c", subcore_axis_name="s"),
        compiler_params=pltpu.CompilerParams(needs_layout_passes=False),
        scratch_types=[
            pltpu.VMEM((EH,), jnp.int32),
            pltpu.VMEM((EH,), jnp.int32),
            pltpu.VMEM((N_NODES,), jnp.float32),
            pltpu.VMEM((N_NODES,), jnp.float32),
            pltpu.VMEM((G, D), jnp.float32),
            pltpu.VMEM((G, D), jnp.float32),
            pltpu.VMEM((G, D), jnp.float32),
            pltpu.VMEM((G, D), jnp.float32),
            pltpu.VMEM((16 + G,), jnp.float32),
            pltpu.VMEM((16 + G,), jnp.float32),
            pltpu.VMEM((G,), jnp.int32),
            pltpu.VMEM((G,), jnp.int32),
            pltpu.VMEM((RB,), jnp.float32),
            pltpu.VMEM_SHARED((N_NODES, D), jnp.float32),
            pltpu.VMEM_SHARED((N_NODES,), jnp.float32),
            pltpu.SemaphoreType.DMA,
            pltpu.SemaphoreType.DMA,
            pltpu.SemaphoreType.DMA,
            pltpu.SemaphoreType.DMA,
            pltpu.SemaphoreType.DMA,
            pltpu.SemaphoreType.DMA,
        ],
    )


FCH = 80


def _fin_body(hp_hbm, den_hbm, out_hbm, a_v, b_v, o_v, d0_v, d1_v, r_v):
    c = lax.axis_index("c")
    s = lax.axis_index("s")
    wid = c * NS + s
    n_chunks = N_NODES // FCH

    def chunk(k, carry):
        g = wid * 4 + k

        @pl.when(g < n_chunks)
        def _do():
            rb = g * FCH
            pltpu.sync_copy(hp_hbm.at[0, pl.ds(rb, FCH)], a_v)
            pltpu.sync_copy(hp_hbm.at[1, pl.ds(rb, FCH)], b_v)
            pltpu.sync_copy(den_hbm.at[pl.ds(rb, FCH)], d0_v)
            pltpu.sync_copy(den_hbm.at[pl.ds(DEN_STRIDE + rb, FCH)], d1_v)
            for q in range(FCH // 16):
                recip = 1.0 / (d0_v[pl.ds(q * 16, 16)]
                               + d1_v[pl.ds(q * 16, 16)] + 1e-16)
                r_v[pl.ds(16 + q * 16, 16)] = recip
            for j in range(FCH):
                rj = plsc.load_gather(
                    r_v, [jnp.full((16,), 16 + j, jnp.int32)])
                for cc in range(8):
                    o_v[j, pl.ds(cc * 16, 16)] = (
                        a_v[j, pl.ds(cc * 16, 16)]
                        + b_v[j, pl.ds(cc * 16, 16)]) * rj
            pltpu.sync_copy(o_v, out_hbm.at[pl.ds(rb, FCH)])

        return carry

    lax.fori_loop(0, 4, chunk, 0)


@functools.lru_cache(maxsize=1)
def _fin_pass_fn():
    return pl.kernel(
        _fin_body,
        out_type=jax.ShapeDtypeStruct((N_NODES, D), jnp.float32),
        mesh=plsc.VectorSubcoreMesh(core_axis_name="c", subcore_axis_name="s"),
        compiler_params=pltpu.CompilerParams(needs_layout_passes=False),
        scratch_types=[
            pltpu.VMEM((FCH, D), jnp.float32),
            pltpu.VMEM((FCH, D), jnp.float32),
            pltpu.VMEM((FCH, D), jnp.float32),
            pltpu.VMEM((FCH,), jnp.float32),
            pltpu.VMEM((FCH,), jnp.float32),
            pltpu.VMEM((16 + FCH,), jnp.float32),
        ],
    )


def kernel(x, edge_index, W_src, W_dst, attn_w):
    z, s3 = _project(x, W_src, W_dst, attn_w)
    ssrc = s3[:, 0, :].reshape(N_NODES)
    sdst = s3[:, 1, :].reshape(N_NODES)
    src = edge_index[0]
    dst = edge_index[1]
    hp, den = _edge_pass_fn()(z, ssrc, sdst, src, dst)
    return _fin_pass_fn()(hp, den)
